# baseline (speedup 1.0000x reference)
import functools

import jax
import jax.numpy as jnp
from jax import lax
from jax.experimental import pallas as pl
from jax.experimental.pallas import tpu as pltpu
from jax.experimental.pallas import tpu_sc as plsc

_K = 32
_NF = 100
_TW = _K + 1


def _sc_gather_one(table, idx):
    info = plsc.get_sparse_core_info()
    nc, ns = info.num_cores, info.num_subcores
    nw = nc * ns
    b = idx.shape[0]
    bpw = b // nw
    mesh = plsc.VectorSubcoreMesh(core_axis_name="c", subcore_axis_name="s")

    @functools.partial(
        pl.kernel,
        mesh=mesh,
        out_type=jax.ShapeDtypeStruct((b, _TW), jnp.float32),
        scratch_types=[
            pltpu.VMEM((bpw,), jnp.int32),
            pltpu.VMEM((bpw, _TW), jnp.float32),
            pltpu.SemaphoreType.DMA,
        ],
    )
    def gather_kernel(t_hbm, i_hbm, o_hbm, idx_v, rows_v, sem):
        wid = lax.axis_index("s") * nc + lax.axis_index("c")
        base = wid * bpw
        pltpu.sync_copy(i_hbm.at[pl.ds(base, bpw)], idx_v)

        def issue_group(g, carry):
            vec = idx_v[pl.ds(g * 16, 16)]
            for k in range(16):
                j = g * 16 + k
                pltpu.async_copy(t_hbm.at[pl.ds(vec[k], 1)],
                                 rows_v.at[pl.ds(j, 1)], sem)
            return carry

        lax.fori_loop(0, bpw // 16, issue_group, 0)
        pltpu.make_async_copy(t_hbm.at[pl.ds(0, bpw)], rows_v, sem).wait()
        pltpu.sync_copy(rows_v, o_hbm.at[pl.ds(base, bpw)])

    return gather_kernel(table, idx)


def _build_weights(feat_table):
    vf = feat_table[:, :_K]
    wf = feat_table[:, _K]
    kx = 2 * _TW + _NF
    ww = jnp.zeros((kx, 2 + _NF), jnp.float32)
    ww = ww.at[_K, 0].set(1.0)
    ww = ww.at[_TW + _K, 1].set(1.0)
    ww = ww.at[2 * _TW + jnp.arange(_NF), 2 + jnp.arange(_NF)].set(wf)
    eye = jnp.eye(_K, dtype=jnp.float32)
    zrow = jnp.zeros((1, _K), jnp.float32)
    ws = jnp.concatenate([eye, zrow, eye, zrow, vf], axis=0)
    wq = jnp.concatenate([eye, zrow, eye, zrow, vf * vf], axis=0)
    return ww, ws, wq


def _fm_body(x_ref, u_ref, i_ref, f_ref, vf_ref, ww_ref, ws_ref, wq_ref,
             v_ref, w_ref, s_ref):
    x = x_ref[...]
    p = lax.Precision.HIGHEST
    w_blk = jnp.dot(x, ww_ref[...], precision=p,
                    preferred_element_type=jnp.float32)
    w_ref[...] = w_blk
    s_sum = jnp.dot(x, ws_ref[...], precision=p,
                    preferred_element_type=jnp.float32)
    s_sq = jnp.dot(x * x, wq_ref[...], precision=p,
                   preferred_element_type=jnp.float32)
    s_ref[...] = (jnp.sum(w_blk, axis=1)
                  + 0.5 * jnp.sum(s_sum * s_sum - s_sq, axis=1))[:, None]
    u = u_ref[...]
    i = i_ref[...]
    f = f_ref[...]
    v_ref[:, 0:1, :] = u[:, :_K][:, None, :]
    v_ref[:, 1:2, :] = i[:, :_K][:, None, :]
    v_ref[:, 2:, :] = f[:, :, None] * vf_ref[...][None, :, :]


def kernel(u, i, feats, user_table, item_table, feat_table, w0):
    b = feats.shape[0]
    u_idx = u.reshape(b).astype(jnp.int32)
    i_idx = i.reshape(b).astype(jnp.int32)
    u_rows = _sc_gather_one(user_table, u_idx)
    i_rows = _sc_gather_one(item_table, i_idx)
    x = jnp.concatenate([u_rows, i_rows, feats], axis=1)
    ww, ws, wq = _build_weights(feat_table)
    vf = feat_table[:, :_K]
    kx = 2 * _TW + _NF
    bb = 256
    v, w, s2 = pl.pallas_call(
        _fm_body,
        grid=(b // bb,),
        in_specs=[
            pl.BlockSpec((bb, kx), lambda g: (g, 0)),
            pl.BlockSpec((bb, _TW), lambda g: (g, 0)),
            pl.BlockSpec((bb, _TW), lambda g: (g, 0)),
            pl.BlockSpec((bb, _NF), lambda g: (g, 0)),
            pl.BlockSpec((_NF, _K), lambda g: (0, 0)),
            pl.BlockSpec((kx, 2 + _NF), lambda g: (0, 0)),
            pl.BlockSpec((kx, _K), lambda g: (0, 0)),
            pl.BlockSpec((kx, _K), lambda g: (0, 0)),
        ],
        out_specs=[
            pl.BlockSpec((bb, 2 + _NF, _K), lambda g: (g, 0, 0)),
            pl.BlockSpec((bb, 2 + _NF), lambda g: (g, 0)),
            pl.BlockSpec((bb, 1), lambda g: (g, 0)),
        ],
        out_shape=[
            jax.ShapeDtypeStruct((b, 2 + _NF, _K), jnp.float32),
            jax.ShapeDtypeStruct((b, 2 + _NF), jnp.float32),
            jax.ShapeDtypeStruct((b, 1), jnp.float32),
        ],
        compiler_params=pltpu.CompilerParams(
            dimension_semantics=("parallel",)),
    )(x, u_rows, i_rows, feats, vf, ww, ws, wq)
    s = s2.reshape(b) + w0
    return (s, w, v)


# speedup vs baseline: 4.9291x; 4.9291x over previous
import functools

import jax
import jax.numpy as jnp
from jax import lax
from jax.experimental import pallas as pl
from jax.experimental.pallas import tpu as pltpu
from jax.experimental.pallas import tpu_sc as plsc

_K = 32
_NF = 100
_TW = _K + 1


def _sc_gather_t(table_t, idx):
    info = plsc.get_sparse_core_info()
    nc, ns = info.num_cores, info.num_subcores
    nw = nc * ns
    b = idx.shape[0]
    bpw = b // nw
    tw = table_t.shape[0]
    mesh = plsc.VectorSubcoreMesh(core_axis_name="c", subcore_axis_name="s")

    ring = 8

    @functools.partial(
        pl.kernel,
        mesh=mesh,
        out_type=jax.ShapeDtypeStruct((tw, b), jnp.float32),
        scratch_types=[
            pltpu.VMEM((bpw + 16,), jnp.int32),
            pltpu.VMEM((tw, bpw), jnp.float32),
        ]
        + [pltpu.VMEM((tw, 128), jnp.float32) for _ in range(ring)]
        + [pltpu.SemaphoreType.DMA for _ in range(ring)],
        compiler_params=pltpu.CompilerParams(needs_layout_passes=False),
    )
    def gather_kernel(t_hbm, i_hbm, o_hbm, idx_v, cols_v, *slots_sems):
        slots = slots_sems[:ring]
        sems = slots_sems[ring:]
        wid = lax.axis_index("s") * nc + lax.axis_index("c")
        base = wid * bpw
        pltpu.sync_copy(i_hbm.at[pl.ds(base, bpw)],
                        idx_v.at[pl.ds(0, bpw)])

        rows0 = lax.broadcasted_iota(jnp.int32, (16,), 0)
        rows1 = rows0 + 16
        row32 = rows0 * 0 + (_K)
        lane0 = rows0 == 0

        def issue(v, k):
            start = pl.multiple_of((v >> 7) << 7, 128)
            pltpu.async_copy(t_hbm.at[:, pl.ds(start, 128)], slots[k],
                             sems[k])

        vec0 = idx_v[pl.ds(0, 16)]
        for k in range(ring):
            issue(vec0[k], k)

        def half_group(h, carry):
            vec = idx_v[pl.ds(h * ring, 16)]
            for k in range(ring):
                j = h * ring + k
                v = vec[k]
                pltpu.make_async_copy(t_hbm.at[:, pl.ds(0, 128)], slots[k],
                                      sems[k]).wait()
                off = v & 127
                offv = rows0 * 0 + off
                c0 = plsc.load_gather(slots[k], [rows0, offv])
                c1 = plsc.load_gather(slots[k], [rows1, offv])
                c2 = plsc.load_gather(slots[k], [row32, offv])
                jv = rows0 * 0 + j
                plsc.store_scatter(cols_v, [rows0, jv], c0)
                plsc.store_scatter(cols_v, [rows1, jv], c1)
                plsc.store_scatter(cols_v, [row32, jv], c2, mask=lane0)

                @pl.when(h < bpw // ring - 1)
                def _():
                    issue(vec[k + ring], k)

            return carry

        lax.fori_loop(0, bpw // ring, half_group, 0)
        pltpu.sync_copy(cols_v, o_hbm.at[:, pl.ds(base, bpw)])

    return gather_kernel(table_t, idx)


def _fm_body_t(u_ref, i_ref, f_ref, vft_ref, vf_ref, wf_ref,
               v_ref, w_ref, s_ref):
    ut = u_ref[...]
    it = i_ref[...]
    ft = f_ref[...]
    uv = ut[:_K, :]
    iv = it[:_K, :]
    uw = ut[_K:_TW, :]
    iw = it[_K:_TW, :]
    vft = vft_ref[...]
    p = lax.Precision.HIGHEST
    s_sum = uv + iv + jnp.dot(vft, ft, precision=p,
                              preferred_element_type=jnp.float32)
    s_sq = (uv * uv + iv * iv
            + jnp.dot(vft * vft, ft * ft, precision=p,
                      preferred_element_type=jnp.float32))
    w_feat = wf_ref[...] * ft
    w_ref[0:1, :] = uw
    w_ref[1:2, :] = iw
    w_ref[2:, :] = w_feat
    s_val = (uw[0, :] + iw[0, :] + jnp.sum(w_feat, axis=0)
             + 0.5 * jnp.sum(s_sum * s_sum - s_sq, axis=0))
    s_ref[...] = s_val[None, :]
    v_ref[0:1, :, :] = uv[None, :, :]
    v_ref[1:2, :, :] = iv[None, :, :]
    v_ref[2:, :, :] = vf_ref[...][:, :, None] * ft[:, None, :]


def kernel(u, i, feats, user_table, item_table, feat_table, w0):
    b = feats.shape[0]
    u_idx = u.reshape(b).astype(jnp.int32)
    i_idx = i.reshape(b).astype(jnp.int32)
    ut_t = _sc_gather_t(user_table.T, u_idx)
    it_t = _sc_gather_t(item_table.T, i_idx)
    f_t = feats.T
    vf = feat_table[:, :_K]
    vft = vf.T
    wf = feat_table[:, _K:_TW]
    bb = 512
    vt, wt, s2 = pl.pallas_call(
        _fm_body_t,
        grid=(b // bb,),
        in_specs=[
            pl.BlockSpec((_TW, bb), lambda g: (0, g)),
            pl.BlockSpec((_TW, bb), lambda g: (0, g)),
            pl.BlockSpec((_NF, bb), lambda g: (0, g)),
            pl.BlockSpec((_K, _NF), lambda g: (0, 0)),
            pl.BlockSpec((_NF, _K), lambda g: (0, 0)),
            pl.BlockSpec((_NF, 1), lambda g: (0, 0)),
        ],
        out_specs=[
            pl.BlockSpec((2 + _NF, _K, bb), lambda g: (0, 0, g)),
            pl.BlockSpec((2 + _NF, bb), lambda g: (0, g)),
            pl.BlockSpec((1, bb), lambda g: (0, g)),
        ],
        out_shape=[
            jax.ShapeDtypeStruct((2 + _NF, _K, b), jnp.float32),
            jax.ShapeDtypeStruct((2 + _NF, b), jnp.float32),
            jax.ShapeDtypeStruct((1, b), jnp.float32),
        ],
        compiler_params=pltpu.CompilerParams(
            dimension_semantics=("parallel",)),
    )(ut_t, it_t, f_t, vft, vf, wf)
    s = s2.reshape(b) + w0
    w = wt.T
    v = vt.transpose(2, 0, 1)
    return (s, w, v)
